# Initial kernel scaffold; baseline (speedup 1.0000x reference)
import jax
import jax.numpy as jnp
from jax.experimental import pallas as pl


def kernel(x, ctrs, band_widths, mags):
    raise NotImplementedError("write your pallas kernel here")


# speedup vs baseline: 8.5460x; 8.5460x over previous
import functools

import jax
import jax.numpy as jnp
import numpy as np
from jax.experimental import pallas as pl
from jax.experimental.pallas import tpu as pltpu

_SUPPORT_P = 0.01
_K2 = float(-np.log(_SUPPORT_P))

_D = 8
_NB = 64
_S = 1024


def _bump_block_kernel(xT_ref, a_ref, b_ref, mags_ref, y_ref):
    q = jnp.zeros((_NB, _S), jnp.float32)
    m = jnp.zeros((_NB, _S), jnp.float32)
    for d in range(_D):
        xd = xT_ref[d : d + 1, :]
        ad = a_ref[:, d : d + 1]
        bd = b_ref[:, d : d + 1]
        z = xd * ad - bd
        z2 = z * z
        q = q + z2
        m = jnp.maximum(m, z2)
    val = mags_ref[:, :] * jnp.exp(-q)
    val = jnp.where(m < _K2, val, 0.0)
    y_ref[:, :] = jnp.sum(val, axis=0, keepdims=True)


@jax.jit
def kernel(x, ctrs, band_widths, mags):
    n = x.shape[0]
    npad = -(-n // _S) * _S
    xp = jnp.pad(x, ((0, npad - n), (0, 0)))
    xT = xp.T
    a = 1.0 / band_widths
    b = ctrs / band_widths
    mags2 = mags.reshape(_NB, 1)

    grid = (npad // _S,)
    y = pl.pallas_call(
        _bump_block_kernel,
        grid=grid,
        in_specs=[
            pl.BlockSpec((_D, _S), lambda i: (0, i)),
            pl.BlockSpec((_NB, _D), lambda i: (0, 0)),
            pl.BlockSpec((_NB, _D), lambda i: (0, 0)),
            pl.BlockSpec((_NB, 1), lambda i: (0, 0)),
        ],
        out_specs=pl.BlockSpec((1, _S), lambda i: (0, i)),
        out_shape=jax.ShapeDtypeStruct((1, npad), jnp.float32),
    )(xT, a, b, mags2)
    return y[0, :n]


# speedup vs baseline: 8.6175x; 1.0084x over previous
import jax
import jax.numpy as jnp
import numpy as np
from jax.experimental import pallas as pl

_SUPPORT_P = 0.01
_K = float(np.sqrt(-np.log(_SUPPORT_P)))
_LOG2E = float(np.log2(np.e))

_D = 8
_NB = 64
_S = 1024


def _bump_block_kernel(xT_ref, w_ref, bias_ref, am_ref, bm_ref, mags_ref, y_ref):
    x = xT_ref[:, :]
    feats = jnp.concatenate([x, x * x], axis=0)
    earg = jax.lax.dot_general(
        w_ref[:, :], feats,
        dimension_numbers=(((1,), (0,)), ((), ())),
        preferred_element_type=jnp.float32,
    ) + bias_ref[:, :]
    e = jnp.exp2(earg)
    m = jnp.zeros((_NB, _S), jnp.float32)
    for d in range(_D):
        t = x[d : d + 1, :] * am_ref[:, d : d + 1] - bm_ref[:, d : d + 1]
        m = jnp.maximum(m, jnp.abs(t))
    e = jnp.where(m < 1.0, e, 0.0)
    y_ref[:, :] = jax.lax.dot_general(
        mags_ref[:, :], e,
        dimension_numbers=(((1,), (0,)), ((), ())),
        preferred_element_type=jnp.float32,
    )


@jax.jit
def kernel(x, ctrs, band_widths, mags):
    n = x.shape[0]
    npad = -(-n // _S) * _S
    xp = jnp.pad(x, ((0, npad - n), (0, 0)))
    xT = xp.T

    a = 1.0 / band_widths
    b = ctrs / band_widths
    w = jnp.concatenate([2.0 * _LOG2E * a * b, -_LOG2E * a * a], axis=1)
    bias = (-_LOG2E * jnp.sum(b * b, axis=1)).reshape(_NB, 1)
    am = a / _K
    bm = b / _K
    mags2 = mags.reshape(1, _NB)

    grid = (npad // _S,)
    y = pl.pallas_call(
        _bump_block_kernel,
        grid=grid,
        in_specs=[
            pl.BlockSpec((_D, _S), lambda i: (0, i)),
            pl.BlockSpec((_NB, 2 * _D), lambda i: (0, 0)),
            pl.BlockSpec((_NB, 1), lambda i: (0, 0)),
            pl.BlockSpec((_NB, _D), lambda i: (0, 0)),
            pl.BlockSpec((_NB, _D), lambda i: (0, 0)),
            pl.BlockSpec((1, _NB), lambda i: (0, 0)),
        ],
        out_specs=pl.BlockSpec((1, _S), lambda i: (0, i)),
        out_shape=jax.ShapeDtypeStruct((1, npad), jnp.float32),
    )(xT, w, bias, am, bm, mags2)
    return y[0, :n]
